# Initial kernel scaffold; baseline (speedup 1.0000x reference)
import jax
import jax.numpy as jnp
from jax.experimental import pallas as pl


def kernel(x, edge_index, edge_attrs, W1, b1, g1, be1, we, W2, b2, g2, be2):
    raise NotImplementedError("write your pallas kernel here")


# speedup vs baseline: 7.4079x; 7.4079x over previous
import functools

import jax
import jax.numpy as jnp
from jax import lax
from jax.experimental import pallas as pl
from jax.experimental.pallas import tpu as pltpu
from jax.experimental.pallas import tpu_sc as plsc

N = 10000
E = 320000
D = 128
NC = 2
NS = 16
NW = NC * NS
EPT = E // NW
CH = 80
NCHUNK = EPT // CH
ROWS_PT = N // NS

def _deg_body(src_hbm, dst_hbm, ew_hbm, out_hbm, idx_v, val_v, ones_v,
              zbuf, stage_v, acc):
    c = lax.axis_index("c")
    s = lax.axis_index("s")
    w = c * NS + s

    zero16 = jnp.zeros((16,), jnp.float32)
    one16 = jnp.ones((16,), jnp.float32)

    def zfill(i, _):
        zbuf[pl.ds(i * 16, 16)] = zero16
        return 0
    lax.fori_loop(0, 2000 // 16, zfill, 0)
    for i in range(CH // 16):
        ones_v[pl.ds(i * 16, 16)] = one16

    @pl.when(s < 15)
    def _():
        pltpu.sync_copy(zbuf, acc.at[pl.ds(s * 2000, 2000)])
    plsc.subcore_barrier()

    off16 = jnp.full((16,), N, jnp.int32)

    def chunk(k, _):
        base = w * EPT + k * CH
        pltpu.sync_copy(ew_hbm.at[pl.ds(base, CH)], val_v)
        pltpu.sync_copy(src_hbm.at[pl.ds(base, CH)], idx_v)
        pltpu.sync_copy(val_v, acc.at[idx_v], add=True)
        pltpu.sync_copy(dst_hbm.at[pl.ds(base, CH)], idx_v)
        for i in range(CH // 16):
            sl = pl.ds(i * 16, 16)
            idx_v[sl] = idx_v[sl] + off16
        pltpu.sync_copy(val_v, acc.at[idx_v], add=True)
        for i in range(CH // 16):
            sl = pl.ds(i * 16, 16)
            idx_v[sl] = idx_v[sl] + off16
        pltpu.sync_copy(ones_v, acc.at[idx_v], add=True)
        return 0
    lax.fori_loop(0, NCHUNK, chunk, 0)

    plsc.subcore_barrier()

    @pl.when(s == 0)
    def _():
        pltpu.sync_copy(acc, stage_v)
        pltpu.sync_copy(stage_v, out_hbm.at[pl.ds(c * 3 * N, 3 * N)])


def _mp_body(src_hbm, dst_hbm, sca_hbm, h_hbm, out_hbm, src_v, dst_v,
             sca_v, rows_v, zrows, acc):
    c = lax.axis_index("c")
    s = lax.axis_index("s")
    w = c * NS + s

    zero16 = jnp.zeros((16,), jnp.float32)

    def zfill(i, _):
        for j in range(D // 16):
            zrows[i, pl.ds(j * 16, 16)] = zero16
        return 0
    lax.fori_loop(0, 200, zfill, 0)

    @pl.when(s < 10)
    def _():
        for i in range(5):
            pltpu.sync_copy(zrows, acc.at[pl.ds(s * 1000 + i * 200, 200), :])
    plsc.subcore_barrier()

    def chunk(k, _):
        base = w * EPT + k * CH
        pltpu.sync_copy(src_hbm.at[pl.ds(base, CH)], src_v)
        pltpu.sync_copy(dst_hbm.at[pl.ds(base, CH)], dst_v)
        pltpu.sync_copy(sca_hbm.at[pl.ds(base, CH)], sca_v)
        pltpu.sync_copy(h_hbm.at[src_v], rows_v)

        def erow(g, _):
            scv16 = sca_v[pl.ds(g * 16, 16)]
            for l in range(16):
                e = g * 16 + l
                scv = jnp.full((16,), scv16[l], jnp.float32)
                for j in range(D // 16):
                    sl = pl.ds(j * 16, 16)
                    rows_v[e, sl] = rows_v[e, sl] * scv
            return 0
        lax.fori_loop(0, CH // 16, erow, 0)

        pltpu.sync_copy(rows_v, acc.at[dst_v], add=True)
        return 0
    lax.fori_loop(0, NCHUNK, chunk, 0)

    plsc.subcore_barrier()

    @pl.when(s < 10)
    def _():
        pltpu.sync_copy(acc.at[pl.ds(s * 1000, 1000), :],
                        out_hbm.at[pl.ds(c * N + s * 1000, 1000), :])


@functools.cache
def _sc_kernels():
    mesh = plsc.VectorSubcoreMesh(core_axis_name="c", subcore_axis_name="s",
                                  num_cores=NC, num_subcores=NS)
    deg_kernel = pl.kernel(
        _deg_body,
        out_type=jax.ShapeDtypeStruct((NC * 3 * N,), jnp.float32),
        mesh=mesh,
        scratch_types=[
            pltpu.VMEM((CH,), jnp.int32),
            pltpu.VMEM((CH,), jnp.float32),
            pltpu.VMEM((CH,), jnp.float32),
            pltpu.VMEM((2000,), jnp.float32),
            pltpu.VMEM((3 * N,), jnp.float32),
            pltpu.VMEM_SHARED((3 * N,), jnp.float32),
        ],
    )
    mp_kernel = pl.kernel(
        _mp_body,
        out_type=jax.ShapeDtypeStruct((NC * N, D), jnp.float32),
        mesh=mesh,
        scratch_types=[
            pltpu.VMEM((CH,), jnp.int32),
            pltpu.VMEM((CH,), jnp.int32),
            pltpu.VMEM((CH,), jnp.float32),
            pltpu.VMEM((CH, D), jnp.float32),
            pltpu.VMEM((200, D), jnp.float32),
            pltpu.VMEM_SHARED((N, D), jnp.float32),
        ],
    )
    return deg_kernel, mp_kernel


def _tcA_body(x_ref, w1_ref, degT_ref, eat_ref, we_ref, h_out, gate_out):
    degT = degT_ref[...]
    dsrc = degT[:, 0:1] + degT[:, 3:4]
    a = lax.rsqrt(jnp.clip(dsrc, 1e-12))
    h = jnp.dot(x_ref[...], w1_ref[...], preferred_element_type=jnp.float32)
    h_out[...] = h * a
    gate_out[...] = jnp.sum(eat_ref[...] * we_ref[...], axis=0)


def _tcB_body(p_ref, degT_ref, b1_ref, g1_ref, be1_ref, w2_ref, h2_out):
    p = p_ref[...]
    agg = p[:N] + p[N:]
    degT = degT_ref[...]
    ddst = degT[:, 1:2] + degT[:, 4:5]
    bfac = lax.rsqrt(jnp.clip(ddst, 1e-12))
    agg = agg * bfac + b1_ref[...]
    mean = jnp.mean(agg, axis=0, keepdims=True)
    var = jnp.mean((agg - mean) ** 2, axis=0, keepdims=True)
    hb = (agg - mean) * lax.rsqrt(var + 1e-5) * g1_ref[...] + be1_ref[...]
    h2_out[...] = jnp.dot(hb, w2_ref[...],
                          preferred_element_type=jnp.float32)


def _tcC_body(p_ref, degT_ref, b2_ref, g2_ref, be2_ref, out_ref):
    p = p_ref[...]
    agg = p[:N] + p[N:]
    degT = degT_ref[...]
    cnt = degT[:, 2:3] + degT[:, 5:6]
    agg = agg / jnp.clip(cnt, 1.0) + b2_ref[...]
    mean = jnp.mean(agg, axis=0, keepdims=True)
    var = jnp.mean((agg - mean) ** 2, axis=0, keepdims=True)
    hb = (agg - mean) * lax.rsqrt(var + 1e-5) * g2_ref[...] + be2_ref[...]
    out_ref[...] = jnp.where(hb > 0, hb, 0.1 * (jnp.exp(hb) - 1.0))


def kernel(x, edge_index, edge_attrs, W1, b1, g1, be1, we, W2, b2, g2, be2):
    src = edge_index[0]
    dst = edge_index[1]
    ew = edge_attrs[:, 1]
    eat = edge_attrs.T.reshape(4, E // D, D)

    deg_kernel, mp_kernel = _sc_kernels()
    degs = deg_kernel(src, dst, ew)
    degT = degs.reshape(2 * 3, N).T

    h1s, gate2d = pl.pallas_call(
        _tcA_body,
        out_shape=[jax.ShapeDtypeStruct((N, D), jnp.float32),
                   jax.ShapeDtypeStruct((E // D, D), jnp.float32)],
    )(x, W1, degT, eat, we.reshape(4, 1, 1))
    gate = gate2d.reshape(E)

    p1 = mp_kernel(src, dst, ew, h1s)

    h2 = pl.pallas_call(
        _tcB_body,
        out_shape=jax.ShapeDtypeStruct((N, D), jnp.float32),
    )(p1, degT, b1.reshape(1, D), g1.reshape(1, D), be1.reshape(1, D), W2)

    p2 = mp_kernel(src, dst, gate, h2)

    out = pl.pallas_call(
        _tcC_body,
        out_shape=jax.ShapeDtypeStruct((N, D), jnp.float32),
    )(p2, degT, b2.reshape(1, D), g2.reshape(1, D), be2.reshape(1, D))
    return out


# speedup vs baseline: 20.9803x; 2.8321x over previous
import functools

import jax
import jax.numpy as jnp
from jax import lax
from jax.experimental import pallas as pl
from jax.experimental.pallas import tpu as pltpu
from jax.experimental.pallas import tpu_sc as plsc

N = 10000
E = 320000
D = 128
NC = 2
NS = 16
NW = NC * NS
EPT = E // NW
CHM = 128
NFULL = 78
TL = EPT - NFULL * CHM
NPAIR = NFULL // 2

DBLK = 2000
NDBLK = EPT // DBLK


def _deg_body(src_hbm, dst_hbm, ew_hbm, out_hbm, bsrc, bdst, bew, accl):
    c = lax.axis_index("c")
    s = lax.axis_index("s")
    w = c * NS + s

    zero16 = jnp.zeros((16,), jnp.float32)
    one16 = jnp.ones((16,), jnp.float32)
    off16 = jnp.full((16,), N, jnp.int32)

    def zfill(i, _):
        accl[pl.ds(i * 16, 16)] = zero16
        return 0
    lax.fori_loop(0, (3 * N) // 16, zfill, 0)

    def blk(b, _):
        base = w * EPT + b * DBLK
        pltpu.sync_copy(src_hbm.at[pl.ds(base, DBLK)], bsrc)
        pltpu.sync_copy(dst_hbm.at[pl.ds(base, DBLK)], bdst)
        pltpu.sync_copy(ew_hbm.at[pl.ds(base, DBLK)], bew)

        def grp(i, _):
            sl = pl.ds(i * 16, 16)
            s16 = bsrc[sl]
            d16 = bdst[sl] + off16
            e16 = bew[sl]
            plsc.addupdate_scatter(accl, [s16], e16)
            plsc.addupdate_scatter(accl, [d16], e16)
            plsc.addupdate_scatter(accl, [d16 + off16], one16)
            return 0
        lax.fori_loop(0, DBLK // 16, grp, 0)
        return 0
    lax.fori_loop(0, NDBLK, blk, 0)

    pltpu.sync_copy(accl, out_hbm.at[pl.ds(w * 3 * N, 3 * N)])


def _mp_body(src_hbm, dst_hbm, sca_hbm, h_hbm, out_hbm,
             srcA, dstA, dstA2, scaA, rowsA,
             srcB, dstB, dstB2, scaB, rowsB,
             srcT, dstT, scaT, rowsT, acc,
             semLA, semLB, semGA, semGB, semSA, semSB, semT):
    c = lax.axis_index("c")
    s = lax.axis_index("s")
    w = c * NS + s
    tbase = w * EPT

    zero16 = jnp.zeros((16,), jnp.float32)

    def zfill(i, _):
        for j in range(D // 16):
            rowsA[i, pl.ds(j * 16, 16)] = zero16
        return 0
    lax.fori_loop(0, CHM, zfill, 0)

    @pl.when(s < 10)
    def _():
        for i in range(7):
            pltpu.async_copy(rowsA,
                             acc.at[pl.ds(s * 1000 + i * CHM, CHM), :],
                             semLA)
        pltpu.async_copy(rowsA.at[pl.ds(0, 104), :],
                         acc.at[pl.ds(s * 1000 + 896, 104), :], semLA)
        for i in range(7):
            pltpu.make_async_copy(
                rowsA, acc.at[pl.ds(s * 1000 + i * CHM, CHM), :],
                semLA).wait()
        pltpu.make_async_copy(rowsA.at[pl.ds(0, 104), :],
                              acc.at[pl.ds(s * 1000 + 896, 104), :],
                              semLA).wait()
    plsc.subcore_barrier()

    def loads_start(k, s_v, d_v, e_v, sem):
        base = tbase + k * CHM
        pltpu.async_copy(src_hbm.at[pl.ds(base, CHM)], s_v, sem)
        pltpu.async_copy(dst_hbm.at[pl.ds(base, CHM)], d_v, sem)
        pltpu.async_copy(sca_hbm.at[pl.ds(base, CHM)], e_v, sem)

    def loads_wait(k, s_v, d_v, e_v, sem):
        base = tbase + k * CHM
        pltpu.make_async_copy(src_hbm.at[pl.ds(base, CHM)], s_v, sem).wait()
        pltpu.make_async_copy(dst_hbm.at[pl.ds(base, CHM)], d_v, sem).wait()
        pltpu.make_async_copy(sca_hbm.at[pl.ds(base, CHM)], e_v, sem).wait()

    def idx_copy(d_v, d2_v):
        for j in range(CHM // 16):
            sl = pl.ds(j * 16, 16)
            d2_v[sl] = d_v[sl]

    def compute(rows, e_v):
        def erow(g, _):
            scv16 = e_v[pl.ds(g * 16, 16)]
            for l in range(16):
                e = g * 16 + l
                scv = jnp.full((16,), scv16[l], jnp.float32)
                for j in range(D // 16):
                    sl = pl.ds(j * 16, 16)
                    rows[e, sl] = rows[e, sl] * scv
            return 0
        lax.fori_loop(0, CHM // 16, erow, 0)

    loads_start(0, srcA, dstA, scaA, semLA)
    loads_wait(0, srcA, dstA, scaA, semLA)
    idx_copy(dstA, dstA2)
    pltpu.async_copy(h_hbm.at[srcA], rowsA, semGA)
    loads_start(1, srcB, dstB, scaB, semLB)

    def pair(i, _):
        k1 = 2 * i + 1
        loads_wait(k1, srcB, dstB, scaB, semLB)

        @pl.when(i > 0)
        def _():
            pltpu.make_async_copy(rowsB, acc.at[dstB2], semSB).wait()
        idx_copy(dstB, dstB2)
        pltpu.make_async_copy(h_hbm.at[srcA], rowsA, semGA).wait()
        pltpu.async_copy(h_hbm.at[srcB], rowsB, semGB)
        compute(rowsA, scaA)
        pltpu.async_copy(rowsA, acc.at[dstA2], semSA, add=True)

        @pl.when(i < NPAIR - 1)
        def _():
            loads_start(k1 + 1, srcA, dstA, scaA, semLA)
        @pl.when(i < NPAIR - 1)
        def _():
            loads_wait(k1 + 1, srcA, dstA, scaA, semLA)
            pltpu.make_async_copy(rowsA, acc.at[dstA2], semSA).wait()
            idx_copy(dstA, dstA2)
            pltpu.async_copy(h_hbm.at[srcA], rowsA, semGA)
        pltpu.make_async_copy(h_hbm.at[srcB], rowsB, semGB).wait()
        compute(rowsB, scaB)
        pltpu.async_copy(rowsB, acc.at[dstB2], semSB, add=True)

        @pl.when(i < NPAIR - 1)
        def _():
            loads_start(k1 + 2, srcB, dstB, scaB, semLB)
        return 0
    lax.fori_loop(0, NPAIR, pair, 0)

    pltpu.make_async_copy(rowsA, acc.at[dstA2], semSA).wait()
    pltpu.make_async_copy(rowsB, acc.at[dstB2], semSB).wait()

    tb = tbase + NFULL * CHM
    pltpu.sync_copy(src_hbm.at[pl.ds(tb, TL)], srcT)
    pltpu.sync_copy(dst_hbm.at[pl.ds(tb, TL)], dstT)
    pltpu.sync_copy(sca_hbm.at[pl.ds(tb, TL)], scaT)
    pltpu.async_copy(h_hbm.at[srcT], rowsT, semT).wait()
    scv16 = scaT[pl.ds(0, 16)]
    for l in range(TL):
        scv = jnp.full((16,), scv16[l], jnp.float32)
        for j in range(D // 16):
            sl = pl.ds(j * 16, 16)
            rowsT[l, sl] = rowsT[l, sl] * scv
    pltpu.sync_copy(rowsT, acc.at[dstT], add=True)

    plsc.subcore_barrier()

    @pl.when(s < 10)
    def _():
        pltpu.sync_copy(acc.at[pl.ds(s * 1000, 1000), :],
                        out_hbm.at[pl.ds(c * N + s * 1000, 1000), :])


@functools.cache
def _sc_kernels():
    mesh = plsc.VectorSubcoreMesh(core_axis_name="c", subcore_axis_name="s",
                                  num_cores=NC, num_subcores=NS)
    deg_kernel = pl.kernel(
        _deg_body,
        out_type=jax.ShapeDtypeStruct((NW * 3 * N,), jnp.float32),
        mesh=mesh,
        compiler_params=pltpu.CompilerParams(needs_layout_passes=False),
        scratch_types=[
            pltpu.VMEM((DBLK,), jnp.int32),
            pltpu.VMEM((DBLK,), jnp.int32),
            pltpu.VMEM((DBLK,), jnp.float32),
            pltpu.VMEM((3 * N,), jnp.float32),
        ],
    )
    mp_kernel = pl.kernel(
        _mp_body,
        out_type=jax.ShapeDtypeStruct((NC * N, D), jnp.float32),
        mesh=mesh,
        scratch_types=[
            pltpu.VMEM((CHM,), jnp.int32),
            pltpu.VMEM((CHM,), jnp.int32),
            pltpu.VMEM((CHM,), jnp.int32),
            pltpu.VMEM((CHM,), jnp.float32),
            pltpu.VMEM((CHM, D), jnp.float32),
            pltpu.VMEM((CHM,), jnp.int32),
            pltpu.VMEM((CHM,), jnp.int32),
            pltpu.VMEM((CHM,), jnp.int32),
            pltpu.VMEM((CHM,), jnp.float32),
            pltpu.VMEM((CHM, D), jnp.float32),
            pltpu.VMEM((TL,), jnp.int32),
            pltpu.VMEM((TL,), jnp.int32),
            pltpu.VMEM((TL,), jnp.float32),
            pltpu.VMEM((TL, D), jnp.float32),
            pltpu.VMEM_SHARED((N, D), jnp.float32),
            pltpu.SemaphoreType.DMA,
            pltpu.SemaphoreType.DMA,
            pltpu.SemaphoreType.DMA,
            pltpu.SemaphoreType.DMA,
            pltpu.SemaphoreType.DMA,
            pltpu.SemaphoreType.DMA,
            pltpu.SemaphoreType.DMA,
        ],
    )
    return deg_kernel, mp_kernel


def _tcA_body(x_ref, w1_ref, degT_ref, eat_ref, we_ref, h_out, gate_out):
    dsrc = jnp.sum(degT_ref[0:N, :], axis=1, keepdims=True)
    a = lax.rsqrt(jnp.clip(dsrc, 1e-12))
    h = jnp.dot(x_ref[...], w1_ref[...], preferred_element_type=jnp.float32)
    h_out[...] = h * a
    gate_out[...] = jnp.sum(eat_ref[...] * we_ref[...], axis=0)


def _tcB_body(p_ref, degT_ref, b1_ref, g1_ref, be1_ref, w2_ref, h2_out):
    p = p_ref[...]
    agg = p[:N] + p[N:]
    ddst = jnp.sum(degT_ref[N:2 * N, :], axis=1, keepdims=True)
    bfac = lax.rsqrt(jnp.clip(ddst, 1e-12))
    agg = agg * bfac + b1_ref[...]
    mean = jnp.mean(agg, axis=0, keepdims=True)
    var = jnp.mean((agg - mean) ** 2, axis=0, keepdims=True)
    hb = (agg - mean) * lax.rsqrt(var + 1e-5) * g1_ref[...] + be1_ref[...]
    h2_out[...] = jnp.dot(hb, w2_ref[...],
                          preferred_element_type=jnp.float32)


def _tcC_body(p_ref, degT_ref, b2_ref, g2_ref, be2_ref, out_ref):
    p = p_ref[...]
    agg = p[:N] + p[N:]
    cnt = jnp.sum(degT_ref[2 * N:3 * N, :], axis=1, keepdims=True)
    agg = agg / jnp.clip(cnt, 1.0) + b2_ref[...]
    mean = jnp.mean(agg, axis=0, keepdims=True)
    var = jnp.mean((agg - mean) ** 2, axis=0, keepdims=True)
    hb = (agg - mean) * lax.rsqrt(var + 1e-5) * g2_ref[...] + be2_ref[...]
    out_ref[...] = jnp.where(hb > 0, hb, 0.1 * (jnp.exp(hb) - 1.0))


def kernel(x, edge_index, edge_attrs, W1, b1, g1, be1, we, W2, b2, g2, be2):
    src = edge_index[0]
    dst = edge_index[1]
    ew = edge_attrs[:, 1]
    eat = edge_attrs.T.reshape(4, E // D, D)

    deg_kernel, mp_kernel = _sc_kernels()
    degs = deg_kernel(src, dst, ew)
    degT = degs.reshape(NW, 3 * N).T

    h1s, gate2d = pl.pallas_call(
        _tcA_body,
        out_shape=[jax.ShapeDtypeStruct((N, D), jnp.float32),
                   jax.ShapeDtypeStruct((E // D, D), jnp.float32)],
    )(x, W1, degT, eat, we.reshape(4, 1, 1))
    gate = gate2d.reshape(E)

    p1 = mp_kernel(src, dst, ew, h1s)

    h2 = pl.pallas_call(
        _tcB_body,
        out_shape=jax.ShapeDtypeStruct((N, D), jnp.float32),
    )(p1, degT, b1.reshape(1, D), g1.reshape(1, D), be1.reshape(1, D), W2)

    p2 = mp_kernel(src, dst, gate, h2)

    out = pl.pallas_call(
        _tcC_body,
        out_shape=jax.ShapeDtypeStruct((N, D), jnp.float32),
    )(p2, degT, b2.reshape(1, D), g2.reshape(1, D), be2.reshape(1, D))
    return out


# speedup vs baseline: 21.1034x; 1.0059x over previous
import functools

import jax
import jax.numpy as jnp
from jax import lax
from jax.experimental import pallas as pl
from jax.experimental.pallas import tpu as pltpu
from jax.experimental.pallas import tpu_sc as plsc

N = 10000
E = 320000
D = 128
NC = 2
NS = 16
NW = NC * NS
EPT = E // NW
CHM = 128
NFULL = 78
TL = EPT - NFULL * CHM
NPAIR = NFULL // 2

DBLK = 2000
NDBLK = EPT // DBLK


def _deg_body(src_hbm, dst_hbm, ew_hbm, out_hbm, bsrc, bdst, bew, accl):
    c = lax.axis_index("c")
    s = lax.axis_index("s")
    w = c * NS + s

    zero16 = jnp.zeros((16,), jnp.float32)
    one16 = jnp.ones((16,), jnp.float32)
    off16 = jnp.full((16,), N, jnp.int32)

    def zfill(i, _):
        accl[pl.ds(i * 16, 16)] = zero16
        return 0
    lax.fori_loop(0, (3 * N) // 16, zfill, 0)

    def blk(b, _):
        base = w * EPT + b * DBLK
        pltpu.sync_copy(src_hbm.at[pl.ds(base, DBLK)], bsrc)
        pltpu.sync_copy(dst_hbm.at[pl.ds(base, DBLK)], bdst)
        pltpu.sync_copy(ew_hbm.at[pl.ds(base, DBLK)], bew)

        def grp(i, _):
            sl = pl.ds(i * 16, 16)
            s16 = bsrc[sl]
            d16 = bdst[sl] + off16
            e16 = bew[sl]
            plsc.addupdate_scatter(accl, [s16], e16)
            plsc.addupdate_scatter(accl, [d16], e16)
            plsc.addupdate_scatter(accl, [d16 + off16], one16)
            return 0
        lax.fori_loop(0, DBLK // 16, grp, 0)
        return 0
    lax.fori_loop(0, NDBLK, blk, 0)

    pltpu.sync_copy(accl, out_hbm.at[pl.ds(w * 3 * N, 3 * N)])


def _mp_body(src_hbm, dst_hbm, sca_hbm, h_hbm, out_hbm,
             srcA, dstA, dstA2, scaA, rowsA,
             srcB, dstB, dstB2, scaB, rowsB,
             srcT, dstT, scaT, rowsT, acc,
             semLA, semLB, semGA, semGB, semSA, semSB, semT):
    c = lax.axis_index("c")
    s = lax.axis_index("s")
    w = c * NS + s
    tbase = w * EPT

    zero16 = jnp.zeros((16,), jnp.float32)

    def zfill(i, _):
        for j in range(D // 16):
            rowsA[i, pl.ds(j * 16, 16)] = zero16
        return 0
    lax.fori_loop(0, CHM, zfill, 0)

    @pl.when(s < 10)
    def _():
        for i in range(7):
            pltpu.async_copy(rowsA,
                             acc.at[pl.ds(s * 1000 + i * CHM, CHM), :],
                             semLA)
        pltpu.async_copy(rowsA.at[pl.ds(0, 104), :],
                         acc.at[pl.ds(s * 1000 + 896, 104), :], semLA)
        for i in range(7):
            pltpu.make_async_copy(
                rowsA, acc.at[pl.ds(s * 1000 + i * CHM, CHM), :],
                semLA).wait()
        pltpu.make_async_copy(rowsA.at[pl.ds(0, 104), :],
                              acc.at[pl.ds(s * 1000 + 896, 104), :],
                              semLA).wait()
    plsc.subcore_barrier()

    def loads_start(k, s_v, d_v, e_v, sem):
        base = tbase + k * CHM
        pltpu.async_copy(src_hbm.at[pl.ds(base, CHM)], s_v, sem)
        pltpu.async_copy(dst_hbm.at[pl.ds(base, CHM)], d_v, sem)
        pltpu.async_copy(sca_hbm.at[pl.ds(base, CHM)], e_v, sem)

    def loads_wait(k, s_v, d_v, e_v, sem):
        base = tbase + k * CHM
        pltpu.make_async_copy(src_hbm.at[pl.ds(base, CHM)], s_v, sem).wait()
        pltpu.make_async_copy(dst_hbm.at[pl.ds(base, CHM)], d_v, sem).wait()
        pltpu.make_async_copy(sca_hbm.at[pl.ds(base, CHM)], e_v, sem).wait()

    def idx_copy(d_v, d2_v):
        for j in range(CHM // 16):
            sl = pl.ds(j * 16, 16)
            d2_v[sl] = d_v[sl]

    def compute(rows, e_v):
        def erow(g, _):
            scv16 = e_v[pl.ds(g * 16, 16)]
            for l in range(16):
                e = g * 16 + l
                scv = jnp.full((16,), scv16[l], jnp.float32)
                for j in range(D // 16):
                    sl = pl.ds(j * 16, 16)
                    rows[e, sl] = rows[e, sl] * scv
            return 0
        lax.fori_loop(0, CHM // 16, erow, 0)

    loads_start(0, srcA, dstA, scaA, semLA)
    loads_wait(0, srcA, dstA, scaA, semLA)
    idx_copy(dstA, dstA2)
    pltpu.async_copy(h_hbm.at[srcA], rowsA, semGA)
    loads_start(1, srcB, dstB, scaB, semLB)

    def pair(i, _):
        k1 = 2 * i + 1
        loads_wait(k1, srcB, dstB, scaB, semLB)

        @pl.when(i > 0)
        def _():
            pltpu.make_async_copy(rowsB, acc.at[dstB2], semSB).wait()
        idx_copy(dstB, dstB2)
        pltpu.make_async_copy(h_hbm.at[srcA], rowsA, semGA).wait()
        pltpu.async_copy(h_hbm.at[srcB], rowsB, semGB)
        compute(rowsA, scaA)
        pltpu.async_copy(rowsA, acc.at[dstA2], semSA, add=True)

        @pl.when(i < NPAIR - 1)
        def _():
            loads_start(k1 + 1, srcA, dstA, scaA, semLA)
        @pl.when(i < NPAIR - 1)
        def _():
            loads_wait(k1 + 1, srcA, dstA, scaA, semLA)
            pltpu.make_async_copy(rowsA, acc.at[dstA2], semSA).wait()
            idx_copy(dstA, dstA2)
            pltpu.async_copy(h_hbm.at[srcA], rowsA, semGA)
        pltpu.make_async_copy(h_hbm.at[srcB], rowsB, semGB).wait()
        compute(rowsB, scaB)
        pltpu.async_copy(rowsB, acc.at[dstB2], semSB, add=True)

        @pl.when(i < NPAIR - 1)
        def _():
            loads_start(k1 + 2, srcB, dstB, scaB, semLB)
        return 0
    lax.fori_loop(0, NPAIR, pair, 0)

    pltpu.make_async_copy(rowsA, acc.at[dstA2], semSA).wait()
    pltpu.make_async_copy(rowsB, acc.at[dstB2], semSB).wait()

    tb = tbase + NFULL * CHM
    pltpu.sync_copy(src_hbm.at[pl.ds(tb, TL)], srcT)
    pltpu.sync_copy(dst_hbm.at[pl.ds(tb, TL)], dstT)
    pltpu.sync_copy(sca_hbm.at[pl.ds(tb, TL)], scaT)
    pltpu.async_copy(h_hbm.at[srcT], rowsT, semT).wait()
    scv16 = scaT[pl.ds(0, 16)]
    for l in range(TL):
        scv = jnp.full((16,), scv16[l], jnp.float32)
        for j in range(D // 16):
            sl = pl.ds(j * 16, 16)
            rowsT[l, sl] = rowsT[l, sl] * scv
    pltpu.sync_copy(rowsT, acc.at[dstT], add=True)

    plsc.subcore_barrier()

    @pl.when(s < 10)
    def _():
        pltpu.sync_copy(acc.at[pl.ds(s * 1000, 1000), :],
                        out_hbm.at[pl.ds(c * N + s * 1000, 1000), :])


@functools.cache
def _sc_kernels():
    mesh = plsc.VectorSubcoreMesh(core_axis_name="c", subcore_axis_name="s",
                                  num_cores=NC, num_subcores=NS)
    deg_kernel = pl.kernel(
        _deg_body,
        out_type=jax.ShapeDtypeStruct((NW * 3 * N,), jnp.float32),
        mesh=mesh,
        compiler_params=pltpu.CompilerParams(needs_layout_passes=False),
        scratch_types=[
            pltpu.VMEM((DBLK,), jnp.int32),
            pltpu.VMEM((DBLK,), jnp.int32),
            pltpu.VMEM((DBLK,), jnp.float32),
            pltpu.VMEM((3 * N,), jnp.float32),
        ],
    )
    mp_kernel = pl.kernel(
        _mp_body,
        out_type=jax.ShapeDtypeStruct((NC * N, D), jnp.float32),
        mesh=mesh,
        scratch_types=[
            pltpu.VMEM((CHM,), jnp.int32),
            pltpu.VMEM((CHM,), jnp.int32),
            pltpu.VMEM((CHM,), jnp.int32),
            pltpu.VMEM((CHM,), jnp.float32),
            pltpu.VMEM((CHM, D), jnp.float32),
            pltpu.VMEM((CHM,), jnp.int32),
            pltpu.VMEM((CHM,), jnp.int32),
            pltpu.VMEM((CHM,), jnp.int32),
            pltpu.VMEM((CHM,), jnp.float32),
            pltpu.VMEM((CHM, D), jnp.float32),
            pltpu.VMEM((TL,), jnp.int32),
            pltpu.VMEM((TL,), jnp.int32),
            pltpu.VMEM((TL,), jnp.float32),
            pltpu.VMEM((TL, D), jnp.float32),
            pltpu.VMEM_SHARED((N, D), jnp.float32),
            pltpu.SemaphoreType.DMA,
            pltpu.SemaphoreType.DMA,
            pltpu.SemaphoreType.DMA,
            pltpu.SemaphoreType.DMA,
            pltpu.SemaphoreType.DMA,
            pltpu.SemaphoreType.DMA,
            pltpu.SemaphoreType.DMA,
        ],
    )
    return deg_kernel, mp_kernel


def _tcA0_body(x_ref, w1_ref, eat_ref, we_ref, h_out, gate_out):
    h = jnp.dot(x_ref[...], w1_ref[...], preferred_element_type=jnp.float32)
    h_out[...] = h
    gate_out[...] = jnp.sum(eat_ref[...] * we_ref[...], axis=0)


def _tcA1_body(h_ref, degT_ref, h_out):
    dsrc = jnp.sum(degT_ref[0:N, :], axis=1, keepdims=True)
    a = lax.rsqrt(jnp.clip(dsrc, 1e-12))
    h_out[...] = h_ref[...] * a


def _tcB_body(p_ref, degT_ref, b1_ref, g1_ref, be1_ref, w2_ref, h2_out):
    p = p_ref[...]
    agg = p[:N] + p[N:]
    ddst = jnp.sum(degT_ref[N:2 * N, :], axis=1, keepdims=True)
    bfac = lax.rsqrt(jnp.clip(ddst, 1e-12))
    agg = agg * bfac + b1_ref[...]
    mean = jnp.mean(agg, axis=0, keepdims=True)
    var = jnp.mean((agg - mean) ** 2, axis=0, keepdims=True)
    hb = (agg - mean) * lax.rsqrt(var + 1e-5) * g1_ref[...] + be1_ref[...]
    h2_out[...] = jnp.dot(hb, w2_ref[...],
                          preferred_element_type=jnp.float32)


def _tcC_body(p_ref, degT_ref, b2_ref, g2_ref, be2_ref, out_ref):
    p = p_ref[...]
    agg = p[:N] + p[N:]
    cnt = jnp.sum(degT_ref[2 * N:3 * N, :], axis=1, keepdims=True)
    agg = agg / jnp.clip(cnt, 1.0) + b2_ref[...]
    mean = jnp.mean(agg, axis=0, keepdims=True)
    var = jnp.mean((agg - mean) ** 2, axis=0, keepdims=True)
    hb = (agg - mean) * lax.rsqrt(var + 1e-5) * g2_ref[...] + be2_ref[...]
    out_ref[...] = jnp.where(hb > 0, hb, 0.1 * (jnp.exp(hb) - 1.0))


def kernel(x, edge_index, edge_attrs, W1, b1, g1, be1, we, W2, b2, g2, be2):
    src = edge_index[0]
    dst = edge_index[1]
    ew = edge_attrs[:, 1]
    eat = edge_attrs.T.reshape(4, E // D, D)

    deg_kernel, mp_kernel = _sc_kernels()
    degs = deg_kernel(src, dst, ew)
    degT = degs.reshape(NW, 3 * N).T

    h1, gate2d = pl.pallas_call(
        _tcA0_body,
        out_shape=[jax.ShapeDtypeStruct((N, D), jnp.float32),
                   jax.ShapeDtypeStruct((E // D, D), jnp.float32)],
    )(x, W1, eat, we.reshape(4, 1, 1))
    gate = gate2d.reshape(E)

    h1s = pl.pallas_call(
        _tcA1_body,
        out_shape=jax.ShapeDtypeStruct((N, D), jnp.float32),
    )(h1, degT)

    p1 = mp_kernel(src, dst, ew, h1s)

    h2 = pl.pallas_call(
        _tcB_body,
        out_shape=jax.ShapeDtypeStruct((N, D), jnp.float32),
    )(p1, degT, b1.reshape(1, D), g1.reshape(1, D), be1.reshape(1, D), W2)

    p2 = mp_kernel(src, dst, gate, h2)

    out = pl.pallas_call(
        _tcC_body,
        out_shape=jax.ShapeDtypeStruct((N, D), jnp.float32),
    )(p2, degT, b2.reshape(1, D), g2.reshape(1, D), be2.reshape(1, D))
    return out
